# baseline (speedup 1.0000x reference)
import jax
import jax.numpy as jnp
from jax import lax
from jax.experimental import pallas as pl
from jax.experimental.pallas import tpu as pltpu

_EPS = 1e-5


def _in_relu(a, g_ref, b_ref, cmask, ho, wp, n):
    c = a.shape[1]
    a3 = a.reshape(ho, wp, c)
    v = jnp.where(cmask, a3, 0.0)
    s = jnp.sum(v, axis=(0, 1), keepdims=True)
    q = jnp.sum(v * v, axis=(0, 1), keepdims=True)
    mean = s / n
    rs = lax.rsqrt(q / n - mean * mean + _EPS)
    sc = rs * g_ref[...].reshape(1, 1, c)
    sh = b_ref[...].reshape(1, 1, c) - mean * sc
    h = jnp.where(cmask, jnp.maximum(a3 * sc + sh, 0.0), 0.0)
    return h.astype(jnp.bfloat16).reshape(ho * wp, c)


def _conv3x3(zb, w_ref, ho, wp):
    c = zb.shape[1]
    m = ho * wp
    z3 = zb.reshape(ho, wp, c)
    zer = jnp.zeros((2, wp, c), zb.dtype)
    rf = jnp.concatenate([zer, z3, zer], axis=0).reshape((ho + 4) * wp, c)
    acc = None
    for dy in range(3):
        for dx in range(3):
            s0 = wp * dy + dx + wp - 1
            t = jnp.dot(rf[s0:s0 + m], w_ref[3 * dy + dx],
                        preferred_element_type=jnp.float32)
            acc = t if acc is None else acc + t
    return acc


def _sample_chain(xs, h_scr, w0, wp_ref, w1, w2, w3,
                  g00, b00, g10, b10, g01, b01, g11, b11, ho, wp,
                  o_ref, slot):
    c0 = xs.shape[0]
    c1 = w0.shape[-1]
    hi = 2 * ho
    m = ho * wp
    nv0 = float(hi * hi)
    nv1 = float(ho * ho)

    cmask = lax.broadcasted_iota(jnp.int32, (ho, wp, 1), 1) < ho

    s = jnp.sum(xs, axis=1, keepdims=True)
    q = jnp.sum(xs * xs, axis=1, keepdims=True)
    mean = s / nv0
    rs = lax.rsqrt(q / nv0 - mean * mean + _EPS)
    sc = rs * jnp.transpose(g00[...])
    sh = jnp.transpose(b00[...]) - mean * sc
    hcm = jnp.maximum(xs * sc + sh, 0.0)

    h_scr[...] = jnp.transpose(hcm).reshape(hi, hi, c0)
    zc = jnp.zeros((ho, wp - ho, c0), jnp.bfloat16)
    zr = jnp.zeros((2, wp, c0), jnp.bfloat16)
    hb = [jnp.concatenate(
              [jnp.concatenate([h_scr[i::2, j::2, :].astype(jnp.bfloat16),
                                zc], axis=1), zr],
              axis=0).reshape((ho + 2) * wp, c0)
          for i in (0, 1) for j in (0, 1)]
    yield

    short = jnp.dot(hb[0][0:m], wp_ref[...],
                    preferred_element_type=jnp.float32)

    acc = None
    for dy in range(3):
        for dx in range(3):
            g = (dy % 2) * 2 + (dx % 2)
            s0 = (dy // 2) * wp + (dx // 2)
            t = jnp.dot(hb[g][s0:s0 + m], w0[3 * dy + dx],
                        preferred_element_type=jnp.float32)
            acc = t if acc is None else acc + t
    yield

    z0 = _in_relu(acc, g10, b10, cmask, ho, wp, nv1)
    yield
    y1 = _conv3x3(z0, w1, ho, wp) + short
    y1 = jnp.where(cmask, y1.reshape(ho, wp, c1), 0.0).reshape(m, c1)
    yield

    h1 = _in_relu(y1, g01, b01, cmask, ho, wp, nv1)
    yield
    acc2 = _conv3x3(h1, w2, ho, wp)
    yield
    z1 = _in_relu(acc2, g11, b11, cmask, ho, wp, nv1)
    yield
    out = _conv3x3(z1, w3, ho, wp) + y1

    outv = out.reshape(ho, wp, c1)[:, :ho, :]
    o_ref[slot] = jnp.transpose(outv, (2, 0, 1))


def _block_group_body(x_ref, w0, wp_ref, w1, w2, w3,
                      g00, b00, g10, b10, g01, b01, g11, b11, o_ref,
                      scr_a, scr_b):
    wp = 32
    ho = o_ref.shape[2]
    args = (w0, wp_ref, w1, w2, w3,
            g00, b00, g10, b10, g01, b01, g11, b11, ho, wp, o_ref)
    ga = _sample_chain(x_ref[0], scr_a, *args, 0)
    gb = _sample_chain(x_ref[1], scr_b, *args, 1)
    order = [ga] + [g for _ in range(7) for g in (ga, gb)] + [gb]
    for g in order:
        next(g, None)


def kernel(x, g0_0, b0_0, w0_0, g1_0, b1_0, w1_0, w_proj_0,
           g0_1, b0_1, w0_1, g1_1, b1_1, w1_1):
    b, c0, h, w = x.shape
    c1 = w0_0.shape[-1]
    ho, wo = h // 2, w // 2
    hp, wp = ho + 2, 32

    xf = x.reshape(b, c0, h * w)

    bf16 = jnp.bfloat16
    wb0 = w0_0.reshape(9, c0, c1).astype(bf16)
    wb1 = w1_0.reshape(9, c1, c1).astype(bf16)
    wb2 = w0_1.reshape(9, c1, c1).astype(bf16)
    wb3 = w1_1.reshape(9, c1, c1).astype(bf16)
    wpb = w_proj_0.astype(bf16)

    vecs = [g1_0, b1_0, g0_1, b0_1, g1_1, b1_1]
    g10, b10, g01, b01, g11, b11 = [v.reshape(1, c1) for v in vecs]
    g00, b00 = g0_0.reshape(1, c0), b0_0.reshape(1, c0)

    w9_spec = lambda c: pl.BlockSpec((9, c, c1), lambda i: (0, 0, 0))
    vec_spec = lambda c: pl.BlockSpec((1, c), lambda i: (0, 0))

    def _launch(xfl, *consts):
        bl = xfl.shape[0]
        return pl.pallas_call(
            _block_group_body,
            out_shape=jax.ShapeDtypeStruct((bl, c1, ho, ho), x.dtype),
            grid_spec=pltpu.PrefetchScalarGridSpec(
                num_scalar_prefetch=0,
                grid=(bl // 2,),
                in_specs=[
                    pl.BlockSpec((2, c0, h * w), lambda i: (i, 0, 0)),
                    w9_spec(c0),
                    pl.BlockSpec((c0, c1), lambda i: (0, 0)),
                    w9_spec(c1), w9_spec(c1), w9_spec(c1),
                    vec_spec(c0), vec_spec(c0),
                    vec_spec(c1), vec_spec(c1), vec_spec(c1),
                    vec_spec(c1), vec_spec(c1), vec_spec(c1),
                ],
                out_specs=pl.BlockSpec((2, c1, ho, ho),
                                       lambda i: (i, 0, 0, 0)),
                scratch_shapes=[pltpu.VMEM((h, w, c0), jnp.float32),
                                pltpu.VMEM((h, w, c0), jnp.float32)],
            ),
            compiler_params=pltpu.CompilerParams(
                dimension_semantics=("parallel",)),
        )(xfl, *consts)

    return _launch(xf, wb0, wpb, wb1, wb2, wb3,
                   g00, b00, g10, b10, g01, b01, g11, b11)


# speedup vs baseline: 1.0618x; 1.0618x over previous
import jax
import jax.numpy as jnp
from jax import lax
from jax.experimental import pallas as pl
from jax.experimental.pallas import tpu as pltpu

_EPS = 1e-5


def _in_relu(a, g_ref, b_ref, cmask, ho, wp, n):
    c = a.shape[1]
    a3 = a.reshape(ho, wp, c)
    v = jnp.where(cmask, a3, 0.0)
    s = jnp.sum(v, axis=(0, 1), keepdims=True)
    q = jnp.sum(v * v, axis=(0, 1), keepdims=True)
    mean = s / n
    rs = lax.rsqrt(q / n - mean * mean + _EPS)
    sc = rs * g_ref[...].reshape(1, 1, c)
    sh = b_ref[...].reshape(1, 1, c) - mean * sc
    h = jnp.where(cmask, jnp.maximum(a3 * sc + sh, 0.0), 0.0)
    return h.astype(jnp.bfloat16).reshape(ho * wp, c)


def _conv3x3(zb, w_ref, ho, wp):
    c = zb.shape[1]
    m = ho * wp
    z3 = zb.reshape(ho, wp, c)
    zer = jnp.zeros((2, wp, c), zb.dtype)
    rf = jnp.concatenate([zer, z3, zer], axis=0).reshape((ho + 4) * wp, c)
    acc = None
    for dy in range(3):
        for dx in range(3):
            s0 = wp * dy + dx + wp - 1
            t = jnp.dot(rf[s0:s0 + m], w_ref[3 * dy + dx],
                        preferred_element_type=jnp.float32)
            acc = t if acc is None else acc + t
    return acc


def _sample_chain(xs, h_scr, w0, wp_ref, w1, w2, w3,
                  g00, b00, g10, b10, g01, b01, g11, b11, ho, wp,
                  o_ref, slot):
    c0 = xs.shape[0]
    c1 = w0.shape[-1]
    hi = 2 * ho
    m = ho * wp
    nv0 = float(hi * hi)
    nv1 = float(ho * ho)

    cmask = lax.broadcasted_iota(jnp.int32, (ho, wp, 1), 1) < ho

    s = jnp.sum(xs, axis=1, keepdims=True)
    q = jnp.sum(xs * xs, axis=1, keepdims=True)
    mean = s / nv0
    rs = lax.rsqrt(q / nv0 - mean * mean + _EPS)
    sc = rs * jnp.transpose(g00[...])
    sh = jnp.transpose(b00[...]) - mean * sc
    hcm = jnp.maximum(xs * sc + sh, 0.0)

    h_scr[...] = jnp.transpose(hcm).reshape(hi, hi, c0)
    zc = jnp.zeros((ho, wp - ho, c0), jnp.bfloat16)
    zr = jnp.zeros((2, wp, c0), jnp.bfloat16)
    hb = [jnp.concatenate(
              [jnp.concatenate([h_scr[i::2, j::2, :].astype(jnp.bfloat16),
                                zc], axis=1), zr],
              axis=0).reshape((ho + 2) * wp, c0)
          for i in (0, 1) for j in (0, 1)]
    yield

    short = jnp.dot(hb[0][0:m], wp_ref[...],
                    preferred_element_type=jnp.float32)

    acc = None
    for dy in range(3):
        for dx in range(3):
            g = (dy % 2) * 2 + (dx % 2)
            s0 = (dy // 2) * wp + (dx // 2)
            t = jnp.dot(hb[g][s0:s0 + m], w0[3 * dy + dx],
                        preferred_element_type=jnp.float32)
            acc = t if acc is None else acc + t
    yield

    z0 = _in_relu(acc, g10, b10, cmask, ho, wp, nv1)
    yield
    y1 = _conv3x3(z0, w1, ho, wp) + short
    y1 = jnp.where(cmask, y1.reshape(ho, wp, c1), 0.0).reshape(m, c1)
    yield

    h1 = _in_relu(y1, g01, b01, cmask, ho, wp, nv1)
    yield
    acc2 = _conv3x3(h1, w2, ho, wp)
    yield
    z1 = _in_relu(acc2, g11, b11, cmask, ho, wp, nv1)
    yield
    out = _conv3x3(z1, w3, ho, wp) + y1

    o_ref[slot] = out.reshape(ho, wp, c1)


def _block_group_body(x_ref, w0, wp_ref, w1, w2, w3,
                      g00, b00, g10, b10, g01, b01, g11, b11, o_ref,
                      scr_a, scr_b):
    ho, wp = o_ref.shape[1], o_ref.shape[2]
    args = (w0, wp_ref, w1, w2, w3,
            g00, b00, g10, b10, g01, b01, g11, b11, ho, wp, o_ref)
    ga = _sample_chain(x_ref[0], scr_a, *args, 0)
    gb = _sample_chain(x_ref[1], scr_b, *args, 1)
    order = [ga] + [g for _ in range(7) for g in (ga, gb)] + [gb]
    for g in order:
        next(g, None)


def kernel(x, g0_0, b0_0, w0_0, g1_0, b1_0, w1_0, w_proj_0,
           g0_1, b0_1, w0_1, g1_1, b1_1, w1_1):
    b, c0, h, w = x.shape
    c1 = w0_0.shape[-1]
    ho, wo = h // 2, w // 2
    hp, wp = ho + 2, 32

    xf = x.reshape(b, c0, h * w)

    bf16 = jnp.bfloat16
    wb0 = w0_0.reshape(9, c0, c1).astype(bf16)
    wb1 = w1_0.reshape(9, c1, c1).astype(bf16)
    wb2 = w0_1.reshape(9, c1, c1).astype(bf16)
    wb3 = w1_1.reshape(9, c1, c1).astype(bf16)
    wpb = w_proj_0.astype(bf16)

    vecs = [g1_0, b1_0, g0_1, b0_1, g1_1, b1_1]
    g10, b10, g01, b01, g11, b11 = [v.reshape(1, c1) for v in vecs]
    g00, b00 = g0_0.reshape(1, c0), b0_0.reshape(1, c0)

    w9_spec = lambda c: pl.BlockSpec((9, c, c1), lambda i: (0, 0, 0))
    vec_spec = lambda c: pl.BlockSpec((1, c), lambda i: (0, 0))

    def _launch(xfl, *consts):
        bl = xfl.shape[0]
        return pl.pallas_call(
            _block_group_body,
            out_shape=jax.ShapeDtypeStruct((bl, ho, wp, c1), x.dtype),
            grid_spec=pltpu.PrefetchScalarGridSpec(
                num_scalar_prefetch=0,
                grid=(bl // 2,),
                in_specs=[
                    pl.BlockSpec((2, c0, h * w), lambda i: (i, 0, 0)),
                    w9_spec(c0),
                    pl.BlockSpec((c0, c1), lambda i: (0, 0)),
                    w9_spec(c1), w9_spec(c1), w9_spec(c1),
                    vec_spec(c0), vec_spec(c0),
                    vec_spec(c1), vec_spec(c1), vec_spec(c1),
                    vec_spec(c1), vec_spec(c1), vec_spec(c1),
                ],
                out_specs=pl.BlockSpec((2, ho, wp, c1),
                                       lambda i: (i, 0, 0, 0)),
                scratch_shapes=[pltpu.VMEM((h, w, c0), jnp.float32),
                                pltpu.VMEM((h, w, c0), jnp.float32)],
            ),
            compiler_params=pltpu.CompilerParams(
                dimension_semantics=("parallel",)),
        )(xfl, *consts)

    out = _launch(xf, wb0, wpb, wb1, wb2, wb3,
                  g00, b00, g10, b10, g01, b01, g11, b11)

    return jnp.transpose(out[:, :, :wo, :], (0, 3, 1, 2))
